# baseline (speedup 1.0000x reference)
import functools

import jax
import jax.numpy as jnp
from jax import lax
from jax.experimental import pallas as pl
from jax.experimental.pallas import tpu as pltpu
from jax.experimental.pallas import tpu_sc as plsc

N = 1000000
D = 64
B = 16384
ENTITY_COFF = 0.1

NC = 2
NS = 16
L = 16
NW = NC * NS
EPW = B // NW
CHUNK = 128
NCHUNK = EPW // CHUNK
GROUPS = CHUNK // L


def _sc_body(table, pairs, xcf_out, xreg_out,
             iu, ip, in_, iep, ien, ru, rp, rn, rep, ren,
             xcf_v, xreg_v, sem):
    wid = lax.axis_index("s") * NC + lax.axis_index("c")
    base = wid * EPW
    idxs = [iu, ip, in_, iep, ien]
    rows = [ru, rp, rn, rep, ren]
    for c in range(NCHUNK):
        cbase = base + c * CHUNK
        for k in range(5):
            pltpu.sync_copy(pairs.at[pl.ds(k * B + cbase, CHUNK)], idxs[k])
        cps = [pltpu.async_copy(table.at[idxs[k]], rows[k], sem)
               for k in range(5)]
        for cp in cps:
            cp.wait()
        for g in range(GROUPS):
            row_ids = lax.iota(jnp.int32, L) + (g * L)

            def dstep(d, carry, row_ids=row_ids):
                acc_cf, acc_reg = carry
                col = jnp.full((L,), 0, jnp.int32) + d
                u = plsc.load_gather(ru, [row_ids, col])
                p = plsc.load_gather(rp, [row_ids, col])
                n = plsc.load_gather(rn, [row_ids, col])
                ep = plsc.load_gather(rep, [row_ids, col])
                en = plsc.load_gather(ren, [row_ids, col])
                acc_cf = acc_cf + u * (p - n)
                acc_reg = acc_reg + (en - ep) * (p + p - ep - en)
                return acc_cf, acc_reg

            zero = jnp.zeros((L,), jnp.float32)
            acc_cf, acc_reg = lax.fori_loop(0, D, dstep, (zero, zero),
                                            unroll=4)
            off = c * CHUNK + g * L
            xcf_v[pl.ds(off, L)] = acc_cf
            xreg_v[pl.ds(off, L)] = acc_reg
    pltpu.sync_copy(xcf_v, xcf_out.at[pl.ds(base, EPW)])
    pltpu.sync_copy(xreg_v, xreg_out.at[pl.ds(base, EPW)])


_sc_pairs_loss = functools.partial(
    pl.kernel,
    mesh=plsc.VectorSubcoreMesh(core_axis_name="c", subcore_axis_name="s"),
    out_type=[
        jax.ShapeDtypeStruct((B,), jnp.float32),
        jax.ShapeDtypeStruct((B,), jnp.float32),
    ],
    scratch_types=[
        pltpu.VMEM((CHUNK,), jnp.int32),
        pltpu.VMEM((CHUNK,), jnp.int32),
        pltpu.VMEM((CHUNK,), jnp.int32),
        pltpu.VMEM((CHUNK,), jnp.int32),
        pltpu.VMEM((CHUNK,), jnp.int32),
        pltpu.VMEM((CHUNK, D), jnp.float32),
        pltpu.VMEM((CHUNK, D), jnp.float32),
        pltpu.VMEM((CHUNK, D), jnp.float32),
        pltpu.VMEM((CHUNK, D), jnp.float32),
        pltpu.VMEM((CHUNK, D), jnp.float32),
        pltpu.VMEM((EPW,), jnp.float32),
        pltpu.VMEM((EPW,), jnp.float32),
        pltpu.SemaphoreType.DMA,
    ],
    compiler_params=pltpu.CompilerParams(
        needs_layout_passes=False, use_tc_tiling_on_sc=False),
)(_sc_body)


def _loss_body(xcf_ref, xreg_ref, out_ref):
    def neg_logsig_sum(x):
        m = jnp.minimum(x, 0.0)
        z = jnp.exp(-jnp.abs(x))
        return jnp.sum(jnp.log1p(z) - m)

    out_ref[0, 0] = (neg_logsig_sum(xcf_ref[...])
                     + ENTITY_COFF * neg_logsig_sum(xreg_ref[...]))


_tc_loss = pl.pallas_call(
    _loss_body,
    out_shape=jax.ShapeDtypeStruct((1, 1), jnp.float32),
    out_specs=pl.BlockSpec(memory_space=pltpu.SMEM),
)


@jax.jit
def kernel(cached_repr, pos_neg_pair_t):
    pairs = pos_neg_pair_t.T.reshape(-1)
    xcf, xreg = _sc_pairs_loss(cached_repr, pairs)
    loss = _tc_loss(xcf.reshape(128, 128), xreg.reshape(128, 128))
    return loss[0, 0]


# speedup vs baseline: 1.1051x; 1.1051x over previous
import functools

import jax
import jax.numpy as jnp
from jax import lax
from jax.experimental import pallas as pl
from jax.experimental.pallas import tpu as pltpu
from jax.experimental.pallas import tpu_sc as plsc

N = 1000000
D = 64
B = 16384
ENTITY_COFF = 0.1

NC = 2
NS = 16
L = 16
NW = NC * NS
EPW = B // NW
CHUNK = 128
NCHUNK = EPW // CHUNK
GROUPS = CHUNK // L


def _sc_body(table, pairs, xcf_out, xreg_out,
             praw, iu, ip, in_, iep, ien,
             ru, rp, rn, rep, ren, xcf_v, xreg_v, sem):
    wid = lax.axis_index("s") * NC + lax.axis_index("c")
    base = wid * EPW
    idxs = [iu, ip, in_, iep, ien]
    rows = [ru, rp, rn, rep, ren]
    lanes = lax.iota(jnp.int32, L)
    for c in range(NCHUNK):
        cbase = base + c * CHUNK
        pltpu.sync_copy(pairs.at[pl.ds(cbase, CHUNK), :], praw)
        for g in range(GROUPS):
            elem = lanes + g * L
            for k in range(5):
                col = plsc.load_gather(praw, [elem, jnp.full((L,), k,
                                                            jnp.int32)])
                idxs[k][pl.ds(g * L, L)] = col
        cps = [pltpu.async_copy(table.at[idxs[k]], rows[k], sem)
               for k in range(5)]
        for cp in cps:
            cp.wait()
        for g in range(GROUPS):
            elem = lanes + g * L

            def dstep(d, carry, elem=elem):
                acc_cf, acc_reg = carry
                col = (lanes + d) & (D - 1)
                u = plsc.load_gather(ru, [elem, col])
                p = plsc.load_gather(rp, [elem, col])
                n = plsc.load_gather(rn, [elem, col])
                ep = plsc.load_gather(rep, [elem, col])
                en = plsc.load_gather(ren, [elem, col])
                acc_cf = acc_cf + u * (p - n)
                acc_reg = acc_reg + (en - ep) * (p + p - ep - en)
                return acc_cf, acc_reg

            zero = jnp.zeros((L,), jnp.float32)
            acc_cf, acc_reg = lax.fori_loop(0, D, dstep, (zero, zero),
                                            unroll=4)
            off = c * CHUNK + g * L
            xcf_v[pl.ds(off, L)] = acc_cf
            xreg_v[pl.ds(off, L)] = acc_reg
    pltpu.sync_copy(xcf_v, xcf_out.at[pl.ds(base, EPW)])
    pltpu.sync_copy(xreg_v, xreg_out.at[pl.ds(base, EPW)])


_sc_pairs_loss = functools.partial(
    pl.kernel,
    mesh=plsc.VectorSubcoreMesh(core_axis_name="c", subcore_axis_name="s"),
    out_type=[
        jax.ShapeDtypeStruct((B,), jnp.float32),
        jax.ShapeDtypeStruct((B,), jnp.float32),
    ],
    scratch_types=[
        pltpu.VMEM((CHUNK, 5), jnp.int32),
        pltpu.VMEM((CHUNK,), jnp.int32),
        pltpu.VMEM((CHUNK,), jnp.int32),
        pltpu.VMEM((CHUNK,), jnp.int32),
        pltpu.VMEM((CHUNK,), jnp.int32),
        pltpu.VMEM((CHUNK,), jnp.int32),
        pltpu.VMEM((CHUNK, D), jnp.float32),
        pltpu.VMEM((CHUNK, D), jnp.float32),
        pltpu.VMEM((CHUNK, D), jnp.float32),
        pltpu.VMEM((CHUNK, D), jnp.float32),
        pltpu.VMEM((CHUNK, D), jnp.float32),
        pltpu.VMEM((EPW,), jnp.float32),
        pltpu.VMEM((EPW,), jnp.float32),
        pltpu.SemaphoreType.DMA,
    ],
    compiler_params=pltpu.CompilerParams(
        needs_layout_passes=False, use_tc_tiling_on_sc=False),
)(_sc_body)


def _loss_body(xcf_ref, xreg_ref, out_ref):
    def neg_logsig_sum(x):
        m = jnp.minimum(x, 0.0)
        z = jnp.exp(-jnp.abs(x))
        return jnp.sum(jnp.log1p(z) - m)

    out_ref[0, 0] = (neg_logsig_sum(xcf_ref[...])
                     + ENTITY_COFF * neg_logsig_sum(xreg_ref[...]))


_tc_loss = pl.pallas_call(
    _loss_body,
    out_shape=jax.ShapeDtypeStruct((1, 1), jnp.float32),
    out_specs=pl.BlockSpec(memory_space=pltpu.SMEM),
)


@jax.jit
def kernel(cached_repr, pos_neg_pair_t):
    xcf, xreg = _sc_pairs_loss(cached_repr, pos_neg_pair_t)
    loss = _tc_loss(xcf.reshape(128, 128), xreg.reshape(128, 128))
    return loss[0, 0]


# speedup vs baseline: 2.5872x; 2.3411x over previous
import functools

import jax
import jax.numpy as jnp
from jax import lax
from jax.experimental import pallas as pl
from jax.experimental.pallas import tpu as pltpu
from jax.experimental.pallas import tpu_sc as plsc

N = 1000000
D = 64
B = 16384
ENTITY_COFF = 0.1

NC = 2
NS = 16
L = 16
DPC = D // NC
EPT = B // NS
GROUPS = EPT // L


def _sc_body(tableT, pairsT, part_out,
             praw, iu, ip, in_, iep, ien, vu, vp, vn, vep, ven,
             acf, arg, row_sh, sem):
    c = lax.axis_index("c")
    s = lax.axis_index("s")
    ebase = s * EPT
    idxs = [iu, ip, in_, iep, ien]
    vals = [vu, vp, vn, vep, ven]
    pltpu.sync_copy(pairsT.at[:, pl.ds(ebase, EPT)], praw)
    for k in range(5):
        for g in range(GROUPS):
            idxs[k][pl.ds(g * L, L)] = praw[k, pl.ds(g * L, L)]
    zeros = jnp.zeros((L,), jnp.float32)
    for g in range(GROUPS):
        acf[pl.ds(g * L, L)] = zeros
        arg[pl.ds(g * L, L)] = zeros
    def dstep(i, carry):
        d = c * DPC + i
        @pl.when(s == 0)
        def _():
            pltpu.sync_copy(tableT.at[pl.ds(d, 1), :], row_sh)
        plsc.subcore_barrier()
        cps = [pltpu.async_copy(row_sh.at[0].at[idxs[k]], vals[k], sem)
               for k in range(5)]
        for cp in cps:
            cp.wait()

        def gstep(g, carry2):
            off = pl.ds(g * L, L)
            u = vu[off]
            p = vp[off]
            n = vn[off]
            ep = vep[off]
            en = ven[off]
            acf[off] += u * (p - n)
            arg[off] += (en - ep) * (p + p - ep - en)
            return carry2

        lax.fori_loop(0, GROUPS, gstep, 0, unroll=4)
        plsc.subcore_barrier()
        return carry

    lax.fori_loop(0, DPC, dstep, 0)
    obase = c * (2 * B) + ebase
    pltpu.sync_copy(acf, part_out.at[pl.ds(obase, EPT)])
    pltpu.sync_copy(arg, part_out.at[pl.ds(obase + B, EPT)])


_sc_dloop = functools.partial(
    pl.kernel,
    mesh=plsc.VectorSubcoreMesh(core_axis_name="c", subcore_axis_name="s"),
    out_type=jax.ShapeDtypeStruct((4 * B,), jnp.float32),
    scratch_types=[
        pltpu.VMEM((5, EPT), jnp.int32),
        pltpu.VMEM((EPT,), jnp.int32),
        pltpu.VMEM((EPT,), jnp.int32),
        pltpu.VMEM((EPT,), jnp.int32),
        pltpu.VMEM((EPT,), jnp.int32),
        pltpu.VMEM((EPT,), jnp.int32),
        pltpu.VMEM((EPT,), jnp.float32),
        pltpu.VMEM((EPT,), jnp.float32),
        pltpu.VMEM((EPT,), jnp.float32),
        pltpu.VMEM((EPT,), jnp.float32),
        pltpu.VMEM((EPT,), jnp.float32),
        pltpu.VMEM((EPT,), jnp.float32),
        pltpu.VMEM((EPT,), jnp.float32),
        pltpu.VMEM_SHARED((1, N), jnp.float32),
        pltpu.SemaphoreType.DMA,
    ],
    compiler_params=pltpu.CompilerParams(needs_layout_passes=False),
)(_sc_body)


def _loss_body(part_ref, out_ref):
    xcf = part_ref[0, :, :] + part_ref[2, :, :]
    xreg = part_ref[1, :, :] + part_ref[3, :, :]

    def neg_logsig_sum(x):
        m = jnp.minimum(x, 0.0)
        z = jnp.exp(-jnp.abs(x))
        return jnp.sum(jnp.log1p(z) - m)

    out_ref[0, 0] = (neg_logsig_sum(xcf)
                     + ENTITY_COFF * neg_logsig_sum(xreg))


_tc_loss = pl.pallas_call(
    _loss_body,
    out_shape=jax.ShapeDtypeStruct((1, 1), jnp.float32),
    out_specs=pl.BlockSpec(memory_space=pltpu.SMEM),
)


@jax.jit
def kernel(cached_repr, pos_neg_pair_t):
    tableT = cached_repr.T
    pairsT = pos_neg_pair_t.T
    part = _sc_dloop(tableT, pairsT)
    loss = _tc_loss(part.reshape(4, 128, 128))
    return loss[0, 0]
